# baseline (speedup 1.0000x reference)
import functools

import jax
import jax.numpy as jnp
from jax import lax
from jax.experimental import pallas as pl
from jax.experimental.pallas import tpu as pltpu
from jax.experimental.pallas import tpu_sc as plsc

B, S, H = 4, 2048, 512
NLEV, NSEG = 16, 8
NG_ROWS = NLEV * NSEG
LANES = 16


def _shifted(x, k, fill):
    pad = jnp.full((B, k), fill, x.dtype)
    return jnp.concatenate([pad, x[:, :-k]], axis=1)


def _prefix(x, op, fill):
    k = 1
    while k < S:
        x = op(x, _shifted(x, k, fill))
        k *= 2
    return x


GRID = 4
RB = S // GRID


def _tables_kernel(tok_ref, pos_ref, nest_ref, seg_ref, wt_ref,
                   p_ref, ng_ref, idx_ref):
    f32 = jnp.float32
    p_ref[...] = jnp.dot(pos_ref[...], wt_ref[0:H, :],
                         preferred_element_type=f32)

    @pl.when(pl.program_id(0) == 0)
    def _():
        n_proj = jnp.dot(nest_ref[...], wt_ref[H:2 * H, :],
                         preferred_element_type=f32)
        g_proj = jnp.dot(seg_ref[...], wt_ref[2 * H:3 * H, :],
                         preferred_element_type=f32)
        rn = lax.broadcasted_iota(jnp.int32, (NG_ROWS, NLEV), 0)
        cn = lax.broadcasted_iota(jnp.int32, (NG_ROWS, NLEV), 1)
        sel_n = ((rn // NSEG) == cn).astype(f32)
        rg = lax.broadcasted_iota(jnp.int32, (NG_ROWS, NSEG), 0)
        cg = lax.broadcasted_iota(jnp.int32, (NG_ROWS, NSEG), 1)
        sel_g = ((rg % NSEG) == cg).astype(f32)
        ng_ref[...] = (jnp.dot(sel_n, n_proj, preferred_element_type=f32)
                       + jnp.dot(sel_g, g_proj, preferred_element_type=f32))

        tok = tok_ref[...]
        is_open = (tok == 40) | (tok == 123) | (tok == 91)
        is_close = (tok == 41) | (tok == 125) | (tok == 93)
        d = jnp.where(is_open, 1, 0) + jnp.where(is_close, -1, 0)
        s_sum = _prefix(d, jnp.add, 0)
        s_min = _prefix(s_sum, jnp.minimum, 2 ** 30)
        level = s_sum - jnp.minimum(s_min, 0)
        nest_idx = jnp.minimum(level, NLEV - 1)
        trig = jnp.where(tok > 39990, 1, 0)
        seg_idx = jnp.bitwise_and(_prefix(trig, jnp.add, 0), NSEG - 1)
        idx_ref[...] = nest_idx * NSEG + seg_idx


def _tables(tok, pos, nest, seg, wt):
    return pl.pallas_call(
        _tables_kernel,
        grid=(GRID,),
        in_specs=[
            pl.BlockSpec((B, S), lambda i: (0, 0)),
            pl.BlockSpec((RB, H), lambda i: (i, 0)),
            pl.BlockSpec((NLEV, H), lambda i: (0, 0)),
            pl.BlockSpec((NSEG, H), lambda i: (0, 0)),
            pl.BlockSpec((3 * H, H), lambda i: (0, 0)),
        ],
        out_specs=(
            pl.BlockSpec((RB, H), lambda i: (i, 0)),
            pl.BlockSpec((NG_ROWS, H), lambda i: (0, 0)),
            pl.BlockSpec((B, S), lambda i: (0, 0)),
        ),
        out_shape=(
            jax.ShapeDtypeStruct((S, H), jnp.float32),
            jax.ShapeDtypeStruct((NG_ROWS, H), jnp.float32),
            jax.ShapeDtypeStruct((B, S), jnp.int32),
        ),
    )(tok, pos, nest, seg, wt)


NBUF = 3
RCH = 8


def _combine(p, ng, idx):
    info = plsc.get_sparse_core_info()
    nw = info.num_cores * info.num_subcores
    ch = S // nw
    nchunks = ch // RCH
    rpc = B * RCH
    mesh = plsc.VectorSubcoreMesh(core_axis_name="c", subcore_axis_name="s")

    @functools.partial(
        pl.kernel, mesh=mesh,
        out_type=jax.ShapeDtypeStruct((B, S, H), jnp.float32),
        scratch_types=[
            pltpu.VMEM((B * ch + LANES,), jnp.int32),
            pltpu.VMEM((NG_ROWS, H), jnp.float32),
            pltpu.VMEM((NBUF, RCH, H), jnp.float32),
            pltpu.VMEM((NBUF, B * RCH, H), jnp.float32),
            pltpu.SemaphoreType.DMA,
            pltpu.SemaphoreType.DMA,
            [pltpu.SemaphoreType.DMA] * NBUF,
            [pltpu.SemaphoreType.DMA] * NBUF,
        ],
    )
    def scatter_combine(p_hbm, ng_hbm, idx_hbm, out_hbm,
                        idx_v, ng_v, p_v, o_v, ngsem, isem, psems, wsems):
        wid = lax.axis_index("s") * info.num_cores + lax.axis_index("c")
        s0 = wid * ch
        ngc = pltpu.async_copy(ng_hbm, ng_v, ngsem)

        def pfill(j):
            return pltpu.async_copy(p_hbm.at[pl.ds(s0 + j * RCH, RCH)],
                                    p_v.at[j % NBUF], psems[j % NBUF])

        pcs = [pfill(0), pfill(1), pfill(2)]
        ics = [pltpu.async_copy(idx_hbm.at[b, pl.ds(s0, ch)],
                                idx_v.at[pl.ds(b * ch, ch)], isem)
               for b in range(B)]
        for c in ics:
            c.wait()
        ngc.wait()
        wcs = [None] * NBUF
        for j in range(nchunks):
            slot = j % NBUF
            off = j * RCH
            pcs[slot].wait()
            if wcs[slot] is not None:
                for c in wcs[slot]:
                    c.wait()

            @plsc.parallel_loop(0, rpc, step=1, unroll=4)
            def body(i):
                b = i >> 3
                r = i & (RCH - 1)
                k = idx_v[pl.ds(b * ch + off + r, LANES)][0]
                for c in range(H // LANES):
                    sl = pl.ds(c * LANES, LANES)
                    o_v[slot, i, sl] = ng_v[k, sl] + p_v[slot, r, sl]
            wcs[slot] = [
                pltpu.async_copy(o_v.at[slot, pl.ds(b * RCH, RCH)],
                                 out_hbm.at[b, pl.ds(s0 + off, RCH)],
                                 wsems[slot])
                for b in range(B)]
            if j + NBUF < nchunks:
                pcs[slot] = pfill(j + NBUF)
        for slot in range(NBUF):
            if wcs[slot] is not None:
                for c in wcs[slot]:
                    c.wait()

    return scatter_combine(p, ng, idx)


def kernel(token_ids, pos_table, nest_table, seg_table, W):
    tok = token_ids.astype(jnp.int32)
    p, ng, idx = _tables(tok, pos_table, nest_table, seg_table, W.T)
    return _combine(p, ng, idx)


# speedup vs baseline: 1.0336x; 1.0336x over previous
import functools

import jax
import jax.numpy as jnp
from jax import lax
from jax.experimental import pallas as pl
from jax.experimental.pallas import tpu as pltpu
from jax.experimental.pallas import tpu_sc as plsc

B, S, H = 4, 2048, 512
NLEV, NSEG = 16, 8
NG_ROWS = NLEV * NSEG
LANES = 16


def _shifted(x, k, fill):
    pad = jnp.full((B, k), fill, x.dtype)
    return jnp.concatenate([pad, x[:, :-k]], axis=1)


def _prefix(x, op, fill):
    k = 1
    while k < S:
        x = op(x, _shifted(x, k, fill))
        k *= 2
    return x


def _tables_kernel(tok_ref, pos_ref, nest_ref, seg_ref, wt_ref,
                   p_ref, ng_ref, idx_ref):
    f32 = jnp.float32
    p_ref[...] = jnp.dot(pos_ref[...], wt_ref[0:H, :],
                         preferred_element_type=f32)
    n_proj = jnp.dot(nest_ref[...], wt_ref[H:2 * H, :],
                     preferred_element_type=f32)
    g_proj = jnp.dot(seg_ref[...], wt_ref[2 * H:3 * H, :],
                     preferred_element_type=f32)
    rn = lax.broadcasted_iota(jnp.int32, (NG_ROWS, NLEV), 0)
    cn = lax.broadcasted_iota(jnp.int32, (NG_ROWS, NLEV), 1)
    sel_n = ((rn // NSEG) == cn).astype(f32)
    rg = lax.broadcasted_iota(jnp.int32, (NG_ROWS, NSEG), 0)
    cg = lax.broadcasted_iota(jnp.int32, (NG_ROWS, NSEG), 1)
    sel_g = ((rg % NSEG) == cg).astype(f32)
    ng_ref[...] = (jnp.dot(sel_n, n_proj, preferred_element_type=f32)
                   + jnp.dot(sel_g, g_proj, preferred_element_type=f32))

    tok = tok_ref[...]
    is_open = (tok == 40) | (tok == 123) | (tok == 91)
    is_close = (tok == 41) | (tok == 125) | (tok == 93)
    d = jnp.where(is_open, 1, 0) + jnp.where(is_close, -1, 0)
    s_sum = _prefix(d, jnp.add, 0)
    s_min = _prefix(s_sum, jnp.minimum, 2 ** 30)
    level = s_sum - jnp.minimum(s_min, 0)
    nest_idx = jnp.minimum(level, NLEV - 1)
    trig = jnp.where(tok > 39990, 1, 0)
    seg_idx = jnp.bitwise_and(_prefix(trig, jnp.add, 0), NSEG - 1)
    idx_ref[...] = nest_idx * NSEG + seg_idx


def _tables(tok, pos, nest, seg, wt):
    return pl.pallas_call(
        _tables_kernel,
        out_shape=(
            jax.ShapeDtypeStruct((S, H), jnp.float32),
            jax.ShapeDtypeStruct((NG_ROWS, H), jnp.float32),
            jax.ShapeDtypeStruct((B, S), jnp.int32),
        ),
    )(tok, pos, nest, seg, wt)


NBUF = 3
RCH = 8


def _combine(p, ng, idx):
    info = plsc.get_sparse_core_info()
    nw = info.num_cores * info.num_subcores
    ch = S // nw
    nchunks = ch // RCH
    rpc = B * RCH
    mesh = plsc.VectorSubcoreMesh(core_axis_name="c", subcore_axis_name="s")

    @functools.partial(
        pl.kernel, mesh=mesh,
        out_type=jax.ShapeDtypeStruct((B, S, H), jnp.float32),
        scratch_types=[
            pltpu.VMEM((B * ch + LANES,), jnp.int32),
            pltpu.VMEM((NG_ROWS, H), jnp.float32),
            pltpu.VMEM((NBUF, RCH, H), jnp.float32),
            pltpu.VMEM((NBUF, B * RCH, H), jnp.float32),
            pltpu.SemaphoreType.DMA,
            pltpu.SemaphoreType.DMA,
            [pltpu.SemaphoreType.DMA] * NBUF,
            [pltpu.SemaphoreType.DMA] * NBUF,
        ],
    )
    def scatter_combine(p_hbm, ng_hbm, idx_hbm, out_hbm,
                        idx_v, ng_v, p_v, o_v, ngsem, isem, psems, wsems):
        wid = lax.axis_index("s") * info.num_cores + lax.axis_index("c")
        s0 = wid * ch
        ngc = pltpu.async_copy(ng_hbm, ng_v, ngsem)

        def pfill(j):
            return pltpu.async_copy(p_hbm.at[pl.ds(s0 + j * RCH, RCH)],
                                    p_v.at[j % NBUF], psems[j % NBUF])

        pcs = [pfill(0), pfill(1), pfill(2)]
        ics = [pltpu.async_copy(idx_hbm.at[b, pl.ds(s0, ch)],
                                idx_v.at[pl.ds(b * ch, ch)], isem)
               for b in range(B)]
        for c in ics:
            c.wait()
        ngc.wait()
        wcs = [None] * NBUF
        for j in range(nchunks):
            slot = j % NBUF
            off = j * RCH
            pcs[slot].wait()
            if wcs[slot] is not None:
                for c in wcs[slot]:
                    c.wait()

            @plsc.parallel_loop(0, rpc, step=1, unroll=4)
            def body(i):
                b = i >> 3
                r = i & (RCH - 1)
                k = idx_v[pl.ds(b * ch + off + r, LANES)][0]
                for c in range(H // LANES):
                    sl = pl.ds(c * LANES, LANES)
                    o_v[slot, i, sl] = ng_v[k, sl] + p_v[slot, r, sl]
            wcs[slot] = [
                pltpu.async_copy(o_v.at[slot, pl.ds(b * RCH, RCH)],
                                 out_hbm.at[b, pl.ds(s0 + off, RCH)],
                                 wsems[slot])
                for b in range(B)]
            if j + NBUF < nchunks:
                pcs[slot] = pfill(j + NBUF)
        for slot in range(NBUF):
            if wcs[slot] is not None:
                for c in wcs[slot]:
                    c.wait()

    return scatter_combine(p, ng, idx)


def kernel(token_ids, pos_table, nest_table, seg_table, W):
    tok = token_ids.astype(jnp.int32)
    p, ng, idx = _tables(tok, pos_table, nest_table, seg_table, W.T)
    return _combine(p, ng, idx)


# speedup vs baseline: 1.1731x; 1.1350x over previous
import functools

import jax
import jax.numpy as jnp
from jax import lax
from jax.experimental import pallas as pl
from jax.experimental.pallas import tpu as pltpu
from jax.experimental.pallas import tpu_sc as plsc

B, S, H = 4, 2048, 512
NLEV, NSEG = 16, 8
NG_ROWS = NLEV * NSEG
LANES = 16


def _shifted(x, k, fill):
    pad = jnp.full((B, k), fill, x.dtype)
    return jnp.concatenate([pad, x[:, :-k]], axis=1)


def _prefix(x, op, fill):
    k = 1
    while k < S:
        x = op(x, _shifted(x, k, fill))
        k *= 2
    return x


def _tables_kernel(tok_ref, pos_ref, nest_ref, seg_ref, wt_ref,
                   p_ref, ng_ref, idx_ref):
    f32 = jnp.float32
    p_ref[...] = jnp.dot(pos_ref[...], wt_ref[0:H, :],
                         preferred_element_type=f32)
    n_proj = jnp.dot(nest_ref[...], wt_ref[H:2 * H, :],
                     preferred_element_type=f32)
    g_proj = jnp.dot(seg_ref[...], wt_ref[2 * H:3 * H, :],
                     preferred_element_type=f32)
    rn = lax.broadcasted_iota(jnp.int32, (NG_ROWS, NLEV), 0)
    cn = lax.broadcasted_iota(jnp.int32, (NG_ROWS, NLEV), 1)
    sel_n = ((rn // NSEG) == cn).astype(f32)
    rg = lax.broadcasted_iota(jnp.int32, (NG_ROWS, NSEG), 0)
    cg = lax.broadcasted_iota(jnp.int32, (NG_ROWS, NSEG), 1)
    sel_g = ((rg % NSEG) == cg).astype(f32)
    ng_ref[...] = (jnp.dot(sel_n, n_proj, preferred_element_type=f32)
                   + jnp.dot(sel_g, g_proj, preferred_element_type=f32))

    tok = tok_ref[...]
    is_open = (tok == 40) | (tok == 123) | (tok == 91)
    is_close = (tok == 41) | (tok == 125) | (tok == 93)
    d = jnp.where(is_open, 1, 0) + jnp.where(is_close, -1, 0)
    s_sum = _prefix(d, jnp.add, 0)
    s_min = _prefix(s_sum, jnp.minimum, 2 ** 30)
    level = s_sum - jnp.minimum(s_min, 0)
    nest_idx = jnp.minimum(level, NLEV - 1)
    trig = jnp.where(tok > 39990, 1, 0)
    seg_idx = jnp.bitwise_and(_prefix(trig, jnp.add, 0), NSEG - 1)
    idx_ref[...] = nest_idx * NSEG + seg_idx


def _tables(tok, pos, nest, seg, wt):
    return pl.pallas_call(
        _tables_kernel,
        out_shape=(
            jax.ShapeDtypeStruct((S, H), jnp.float32),
            jax.ShapeDtypeStruct((NG_ROWS, H), jnp.float32),
            jax.ShapeDtypeStruct((B, S), jnp.int32),
        ),
    )(tok, pos, nest, seg, wt)


NBUF = 3
OBUF = 2
RCH = 8


def _combine(p, ng, idx):
    info = plsc.get_sparse_core_info()
    nw = info.num_cores * info.num_subcores
    ch = S // nw
    nchunks = ch // RCH
    rpc = B * RCH
    mesh = plsc.VectorSubcoreMesh(core_axis_name="c", subcore_axis_name="s")

    @functools.partial(
        pl.kernel, mesh=mesh,
        out_type=jax.ShapeDtypeStruct((B, S, H), jnp.float32),
        scratch_types=[
            pltpu.VMEM((B * ch + LANES,), jnp.int32),
            pltpu.VMEM((NG_ROWS, H), jnp.float32),
            pltpu.VMEM_SHARED((NG_ROWS, H), jnp.float32),
            pltpu.VMEM((NBUF, RCH, H), jnp.float32),
            pltpu.VMEM((OBUF, B * RCH, H), jnp.float32),
            pltpu.SemaphoreType.DMA,
            pltpu.SemaphoreType.DMA,
            [pltpu.SemaphoreType.DMA] * NBUF,
            [pltpu.SemaphoreType.DMA] * OBUF,
        ],
    )
    def scatter_combine(p_hbm, ng_hbm, idx_hbm, out_hbm,
                        idx_v, ng_v, ng_sh, p_v, o_v, ngsem, isem, psems, wsems):
        wid = lax.axis_index("s") * info.num_cores + lax.axis_index("c")
        s0 = wid * ch

        @pl.when(lax.axis_index("s") == 0)
        def _():
            pltpu.sync_copy(ng_hbm, ng_sh)

        def pfill(j):
            return pltpu.async_copy(p_hbm.at[pl.ds(s0 + j * RCH, RCH)],
                                    p_v.at[j % NBUF], psems[j % NBUF])

        pcs = [pfill(0), pfill(1), pfill(2)]
        ics = [pltpu.async_copy(idx_hbm.at[b, pl.ds(s0, ch)],
                                idx_v.at[pl.ds(b * ch, ch)], isem)
               for b in range(B)]
        plsc.subcore_barrier()
        ngc = pltpu.async_copy(ng_sh, ng_v, ngsem)
        for c in ics:
            c.wait()
        ngc.wait()
        wcs = [None] * OBUF
        for j in range(nchunks):
            slot = j % NBUF
            oslot = j % OBUF
            off = j * RCH
            pcs[slot].wait()
            if wcs[oslot] is not None:
                for c in wcs[oslot]:
                    c.wait()

            @plsc.parallel_loop(0, rpc, step=1, unroll=4)
            def body(i):
                b = i >> 3
                r = i & (RCH - 1)
                k = idx_v[pl.ds(b * ch + off + r, LANES)][0]
                for c in range(H // LANES):
                    sl = pl.ds(c * LANES, LANES)
                    o_v[oslot, i, sl] = ng_v[k, sl] + p_v[slot, r, sl]
            wcs[oslot] = [
                pltpu.async_copy(o_v.at[oslot, pl.ds(b * RCH, RCH)],
                                 out_hbm.at[b, pl.ds(s0 + off, RCH)],
                                 wsems[oslot])
                for b in range(B)]
            if j + NBUF < nchunks:
                pcs[slot] = pfill(j + NBUF)
        for oslot in range(OBUF):
            if wcs[oslot] is not None:
                for c in wcs[oslot]:
                    c.wait()

    return scatter_combine(p, ng, idx)


def kernel(token_ids, pos_table, nest_table, seg_table, W):
    tok = token_ids.astype(jnp.int32)
    p, ng, idx = _tables(tok, pos_table, nest_table, seg_table, W.T)
    return _combine(p, ng, idx)
